# baseline (speedup 1.0000x reference)
import functools

import jax
import jax.numpy as jnp
import numpy as np
from jax import lax
from jax.experimental import pallas as pl
from jax.experimental.pallas import tpu as pltpu
from jax.experimental.pallas import tpu_sc as plsc

_OUT = 7
_C = 256
_HW = (200, 100, 50, 25)
_BASES = (0, 40000, 50000, 52500)
_TABLE_ROWS = 53152
_N_ROI = 1000
_N_PAD = 1024
_NC = 2
_NS = 16
_NW = _NC * _NS
_RPW = _N_PAD // _NW
_T2 = tuple(float((56.0 * (2.0 ** k - 1e-6)) ** 2) for k in (1, 2, 3))

_mesh = plsc.VectorSubcoreMesh(core_axis_name="c", subcore_axis_name="s")


def _sel_lvl(lvl, vals, dtype):
    v = jnp.full((16,), vals[3], dtype)
    v = jnp.where(lvl == 2, jnp.full((16,), vals[2], dtype), v)
    v = jnp.where(lvl == 1, jnp.full((16,), vals[1], dtype), v)
    v = jnp.where(lvl == 0, jnp.full((16,), vals[0], dtype), v)
    return v


_GDN = lax.GatherDimensionNumbers(
    offset_dims=(), collapsed_slice_dims=(0,), start_index_map=(0,))


def _bcast_lane(vec, j):
    lane = jnp.full((16, 1), j, jnp.int32)
    return lax.gather(vec, lane, _GDN, (1,),
                      mode=lax.GatherScatterMode.PROMISE_IN_BOUNDS)


@functools.partial(
    pl.kernel,
    out_type=jax.ShapeDtypeStruct((_N_ROI * 98, 128), jnp.float32),
    mesh=_mesh,
    scratch_types=[
        pltpu.VMEM((_RPW,), jnp.float32),
        pltpu.VMEM((_RPW,), jnp.float32),
        pltpu.VMEM((_RPW,), jnp.float32),
        pltpu.VMEM((_RPW,), jnp.float32),
        pltpu.VMEM((208,), jnp.int32),
        pltpu.VMEM((208,), jnp.float32),
        pltpu.VMEM((208,), jnp.int32),
        pltpu.VMEM((208,), jnp.float32),
        pltpu.VMEM((200, 128), jnp.int32),
        pltpu.VMEM((200, 128), jnp.int32),
        pltpu.VMEM((392, 128), jnp.float32),
        pltpu.SemaphoreType.DMA,
        pltpu.SemaphoreType.DMA,
        pltpu.SemaphoreType.DMA,
    ],
    compiler_params=pltpu.CompilerParams(needs_layout_passes=False),
)
def _sc_roi_align(table, rx1, ry1, rx2, ry2, out,
                  x1v, y1v, x2v, y2v, idx0, w0, idx1, w1,
                  rows0, rows1, outv, semg0, semg1, semo):
    wid = lax.axis_index("s") * _NC + lax.axis_index("c")
    base = wid * _RPW
    pltpu.sync_copy(rx1.at[pl.ds(base, _RPW)], x1v)
    pltpu.sync_copy(ry1.at[pl.ds(base, _RPW)], y1v)
    pltpu.sync_copy(rx2.at[pl.ds(base, _RPW)], x2v)
    pltpu.sync_copy(ry2.at[pl.ds(base, _RPW)], y2v)
    for st in (idx0, idx1):
        st[pl.ds(192, 16)] = jnp.zeros((16,), jnp.int32)
    for st in (w0, w1):
        st[pl.ds(192, 16)] = jnp.zeros((16,), jnp.float32)

    def gen(r, idxv, wv):
        c16 = r // 16
        j = r - c16 * 16
        bx1 = _bcast_lane(x1v[pl.ds(c16 * 16, 16)], j)
        by1 = _bcast_lane(y1v[pl.ds(c16 * 16, 16)], j)
        bx2 = _bcast_lane(x2v[pl.ds(c16 * 16, 16)], j)
        by2 = _bcast_lane(y2v[pl.ds(c16 * 16, 16)], j)
        area = (bx2 - bx1 + 1.0) * (by2 - by1 + 1.0)
        lvl = (jnp.where(area >= _T2[0], 1, 0)
               + jnp.where(area >= _T2[1], 1, 0)
               + jnp.where(area >= _T2[2], 1, 0))
        ss = _sel_lvl(lvl, (0.25, 0.125, 0.0625, 0.03125), jnp.float32)
        dmax = _sel_lvl(lvl, (199.0, 99.0, 49.0, 24.0), jnp.float32)
        wrow = _sel_lvl(lvl, _HW, jnp.int32)
        bofs = _sel_lvl(lvl, _BASES, jnp.int32)
        x1s = bx1 * ss
        y1s = by1 * ss
        roiw = jnp.maximum(bx2 * ss - x1s, 1.0)
        roih = jnp.maximum(by2 * ss - y1s, 1.0)
        binw = roiw / 7.0
        binh = roih / 7.0
        zero = jnp.zeros((16,), jnp.float32)
        for k in range(4):
            p = lax.iota(jnp.int32, 16) + (16 * k)
            pf = p.astype(jnp.float32)
            py = ((pf + 0.5) * (1.0 / 7.0)).astype(jnp.int32)
            px = p - py * 7
            yy = y1s + (py.astype(jnp.float32) + 0.5) * binh
            xx = x1s + (px.astype(jnp.float32) + 0.5) * binw
            yy = jnp.minimum(jnp.maximum(yy, zero), dmax)
            xx = jnp.minimum(jnp.maximum(xx, zero), dmax)
            y0 = yy.astype(jnp.int32)
            x0 = xx.astype(jnp.int32)
            ly = yy - y0.astype(jnp.float32)
            lx = xx - x0.astype(jnp.float32)
            hy = 1.0 - ly
            hx = 1.0 - lx
            i00 = bofs + y0 * wrow + x0
            mask = p < 49
            pc = jnp.where(mask, p, 0)
            taps = (i00, i00 + 1, i00 + wrow, i00 + wrow + 1)
            wts = (hy * hx, hy * lx, ly * hx, ly * lx)
            for t in range(4):
                plsc.store_scatter(idxv, [pc + (49 * t)], taps[t], mask=mask)
                plsc.store_scatter(wv, [pc * 4 + t], wts[t], mask=mask)

    def fire(idxv, rowsv, sem):
        pltpu.async_copy(table.at[idxv.at[pl.ds(0, 96)]],
                         rowsv.at[pl.ds(0, 96)], sem)
        pltpu.async_copy(table.at[idxv.at[pl.ds(96, 104)]],
                         rowsv.at[pl.ds(96, 104)], sem)

    def drain(idxv, rowsv, sem):
        pltpu.make_async_copy(table.at[idxv.at[pl.ds(0, 96)]],
                              rowsv.at[pl.ds(0, 96)], sem).wait()
        pltpu.make_async_copy(table.at[idxv.at[pl.ds(96, 104)]],
                              rowsv.at[pl.ds(96, 104)], sem).wait()

    def compute_out(r, rowsv, wv, m):
        himask = jnp.full((16,), -65536, jnp.int32)
        ce49 = [(lax.iota(jnp.int32, 16) * 2 + 32 * cc) * 49
                for cc in range(8)]

        def point_block(pt, wvec, q):
            b0 = _bcast_lane(wvec, 4 * q)
            b1 = _bcast_lane(wvec, 4 * q + 1)
            b2 = _bcast_lane(wvec, 4 * q + 2)
            b3 = _bcast_lane(wvec, 4 * q + 3)
            ofs = pt + (m * 12544)
            for cc in range(8):
                sl = pl.ds(cc * 16, 16)
                w0v = rowsv[pt, sl]
                w1v = rowsv[pt + 49, sl]
                w2v = rowsv[pt + 98, sl]
                w3v = rowsv[pt + 147, sl]

                def lo(w):
                    return plsc.bitcast(w << 16, jnp.float32)

                def hi(w):
                    return plsc.bitcast(w & himask, jnp.float32)

                ae = ((lo(w0v) * b0 + lo(w1v) * b1)
                      + (lo(w2v) * b2 + lo(w3v) * b3))
                ao = ((hi(w0v) * b0 + hi(w1v) * b1)
                      + (hi(w2v) * b2 + hi(w3v) * b3))
                je = ce49[cc] + ofs
                jo = je + 49
                plsc.store_scatter(
                    outv,
                    [lax.shift_right_logical(je, 7),
                     lax.bitwise_and(je, 127)], ae)
                plsc.store_scatter(
                    outv,
                    [lax.shift_right_logical(jo, 7),
                     lax.bitwise_and(jo, 127)], ao)

        def grp_body(g, carry2):
            wvec = wv[pl.ds(g * 16, 16)]
            for q in range(4):
                point_block(g * 4 + q, wvec, q)
            return carry2

        lax.fori_loop(0, 12, grp_body, 0)
        point_block(48, wv[pl.ds(192, 16)], 0)

    gen(0, idx0, w0)
    fire(idx0, rows0, semg0)

    def quad_body(qi, carry):
        a = 4 * qi
        pl.when(jnp.logical_and(qi > 0, base + a <= _N_ROI))(
            lambda: pltpu.make_async_copy(
                outv, out.at[pl.ds(0, 392)], semo).wait())
        gen(a + 1, idx1, w1)
        fire(idx1, rows1, semg1)
        drain(idx0, rows0, semg0)
        compute_out(a, rows0, w0, 0)
        gen(a + 2, idx0, w0)
        fire(idx0, rows0, semg0)
        drain(idx1, rows1, semg1)
        compute_out(a + 1, rows1, w1, 1)
        gen(a + 3, idx1, w1)
        fire(idx1, rows1, semg1)
        drain(idx0, rows0, semg0)
        compute_out(a + 2, rows0, w0, 2)

        @pl.when(qi < (_RPW // 4 - 1))
        def _():
            gen(a + 4, idx0, w0)
            fire(idx0, rows0, semg0)

        drain(idx1, rows1, semg1)
        compute_out(a + 3, rows1, w1, 3)

        @pl.when(base + a < _N_ROI)
        def _():
            pltpu.async_copy(outv, out.at[pl.ds((base + a) * 98, 392)], semo)

        return carry

    lax.fori_loop(0, _RPW // 4, quad_body, 0)
    pl.when(base + _RPW - 4 < _N_ROI)(lambda: pltpu.make_async_copy(
        outv, out.at[pl.ds(0, 392)], semo).wait())


def kernel(feats_0, feats_1, feats_2, feats_3, rois):
    feats = (feats_0, feats_1, feats_2, feats_3)
    tabs = [f[0].astype(jnp.bfloat16).transpose(1, 2, 0).reshape(-1, _C)
            for f in feats]
    tabs.append(jnp.zeros((_TABLE_ROWS - 53125, _C), jnp.bfloat16))
    table = jnp.concatenate(tabs, axis=0)
    table = lax.bitcast_convert_type(
        table.reshape(_TABLE_ROWS, 128, 2), jnp.int32)
    pad = jnp.zeros((_N_PAD - _N_ROI,), jnp.float32)
    cols = [jnp.concatenate([rois[:, i], pad]) for i in (1, 2, 3, 4)]
    out_rows = _sc_roi_align(table, cols[0], cols[1], cols[2], cols[3])
    return out_rows.reshape(_N_ROI, _C, _OUT, _OUT)


# speedup vs baseline: 2.0496x; 2.0496x over previous
import functools

import jax
import jax.numpy as jnp
import numpy as np
from jax import lax
from jax.experimental import pallas as pl
from jax.experimental.pallas import tpu as pltpu
from jax.experimental.pallas import tpu_sc as plsc

_OUT = 7
_C = 256
_HW = (200, 100, 50, 25)
_BASES = (0, 40000, 50000, 52500)
_TABLE_ROWS = 53152
_N_ROI = 1000
_N_PAD = 1024
_NC = 2
_NS = 16
_NW = _NC * _NS
_RPW = _N_PAD // _NW
_T2 = tuple(float((56.0 * (2.0 ** k - 1e-6)) ** 2) for k in (1, 2, 3))

_mesh = plsc.VectorSubcoreMesh(core_axis_name="c", subcore_axis_name="s")


def _sel_lvl(lvl, vals, dtype):
    v = jnp.full((16,), vals[3], dtype)
    v = jnp.where(lvl == 2, jnp.full((16,), vals[2], dtype), v)
    v = jnp.where(lvl == 1, jnp.full((16,), vals[1], dtype), v)
    v = jnp.where(lvl == 0, jnp.full((16,), vals[0], dtype), v)
    return v


_GDN = lax.GatherDimensionNumbers(
    offset_dims=(), collapsed_slice_dims=(0,), start_index_map=(0,))


def _bcast_lane(vec, j):
    lane = jnp.full((16, 1), j, jnp.int32)
    return lax.gather(vec, lane, _GDN, (1,),
                      mode=lax.GatherScatterMode.PROMISE_IN_BOUNDS)


@functools.partial(
    pl.kernel,
    out_type=jax.ShapeDtypeStruct((_N_PAD * 56, _C), jnp.float32),
    mesh=_mesh,
    scratch_types=[
        pltpu.VMEM((_RPW,), jnp.float32),
        pltpu.VMEM((_RPW,), jnp.float32),
        pltpu.VMEM((_RPW,), jnp.float32),
        pltpu.VMEM((_RPW,), jnp.float32),
        pltpu.VMEM((208,), jnp.int32),
        pltpu.VMEM((208,), jnp.float32),
        pltpu.VMEM((208,), jnp.int32),
        pltpu.VMEM((208,), jnp.float32),
        pltpu.VMEM((200, 128), jnp.int32),
        pltpu.VMEM((200, 128), jnp.int32),
        pltpu.VMEM((56, _C), jnp.float32),
        pltpu.SemaphoreType.DMA,
        pltpu.SemaphoreType.DMA,
        pltpu.SemaphoreType.DMA,
    ],
    compiler_params=pltpu.CompilerParams(needs_layout_passes=False),
)
def _sc_roi_align(table, rx1, ry1, rx2, ry2, out,
                  x1v, y1v, x2v, y2v, idx0, w0, idx1, w1,
                  rows0, rows1, outv, semg0, semg1, semo):
    wid = lax.axis_index("s") * _NC + lax.axis_index("c")
    base = wid * _RPW
    pltpu.sync_copy(rx1.at[pl.ds(base, _RPW)], x1v)
    pltpu.sync_copy(ry1.at[pl.ds(base, _RPW)], y1v)
    pltpu.sync_copy(rx2.at[pl.ds(base, _RPW)], x2v)
    pltpu.sync_copy(ry2.at[pl.ds(base, _RPW)], y2v)
    for st in (idx0, idx1):
        st[pl.ds(192, 16)] = jnp.zeros((16,), jnp.int32)
    for st in (w0, w1):
        st[pl.ds(192, 16)] = jnp.zeros((16,), jnp.float32)

    def gen(r, idxv, wv):
        c16 = r // 16
        j = r - c16 * 16
        bx1 = _bcast_lane(x1v[pl.ds(c16 * 16, 16)], j)
        by1 = _bcast_lane(y1v[pl.ds(c16 * 16, 16)], j)
        bx2 = _bcast_lane(x2v[pl.ds(c16 * 16, 16)], j)
        by2 = _bcast_lane(y2v[pl.ds(c16 * 16, 16)], j)
        area = (bx2 - bx1 + 1.0) * (by2 - by1 + 1.0)
        lvl = (jnp.where(area >= _T2[0], 1, 0)
               + jnp.where(area >= _T2[1], 1, 0)
               + jnp.where(area >= _T2[2], 1, 0))
        ss = _sel_lvl(lvl, (0.25, 0.125, 0.0625, 0.03125), jnp.float32)
        dmax = _sel_lvl(lvl, (199.0, 99.0, 49.0, 24.0), jnp.float32)
        wrow = _sel_lvl(lvl, _HW, jnp.int32)
        bofs = _sel_lvl(lvl, _BASES, jnp.int32)
        x1s = bx1 * ss
        y1s = by1 * ss
        roiw = jnp.maximum(bx2 * ss - x1s, 1.0)
        roih = jnp.maximum(by2 * ss - y1s, 1.0)
        binw = roiw / 7.0
        binh = roih / 7.0
        zero = jnp.zeros((16,), jnp.float32)
        for k in range(4):
            p = lax.iota(jnp.int32, 16) + (16 * k)
            pf = p.astype(jnp.float32)
            py = ((pf + 0.5) * (1.0 / 7.0)).astype(jnp.int32)
            px = p - py * 7
            yy = y1s + (py.astype(jnp.float32) + 0.5) * binh
            xx = x1s + (px.astype(jnp.float32) + 0.5) * binw
            yy = jnp.minimum(jnp.maximum(yy, zero), dmax)
            xx = jnp.minimum(jnp.maximum(xx, zero), dmax)
            y0 = yy.astype(jnp.int32)
            x0 = xx.astype(jnp.int32)
            ly = yy - y0.astype(jnp.float32)
            lx = xx - x0.astype(jnp.float32)
            hy = 1.0 - ly
            hx = 1.0 - lx
            i00 = bofs + y0 * wrow + x0
            mask = p < 49
            pc = jnp.where(mask, p, 0)
            taps = (i00, i00 + 1, i00 + wrow, i00 + wrow + 1)
            wts = (hy * hx, hy * lx, ly * hx, ly * lx)
            for t in range(4):
                plsc.store_scatter(idxv, [pc + (49 * t)], taps[t], mask=mask)
                plsc.store_scatter(wv, [pc * 4 + t], wts[t], mask=mask)

    def fire(idxv, rowsv, sem):
        pltpu.async_copy(table.at[idxv.at[pl.ds(0, 96)]],
                         rowsv.at[pl.ds(0, 96)], sem)
        pltpu.async_copy(table.at[idxv.at[pl.ds(96, 104)]],
                         rowsv.at[pl.ds(96, 104)], sem)

    def drain(idxv, rowsv, sem):
        pltpu.make_async_copy(table.at[idxv.at[pl.ds(0, 96)]],
                              rowsv.at[pl.ds(0, 96)], sem).wait()
        pltpu.make_async_copy(table.at[idxv.at[pl.ds(96, 104)]],
                              rowsv.at[pl.ds(96, 104)], sem).wait()

    def compute_out(r, rowsv, wv, first):
        if first:
            pl.when(r > 0)(lambda: pltpu.make_async_copy(
                outv, out.at[pl.ds(0, 56)], semo).wait())
        else:
            pltpu.make_async_copy(outv, out.at[pl.ds(0, 56)], semo).wait()

        himask = jnp.full((16,), -65536, jnp.int32)
        two_iota = lax.iota(jnp.int32, 16) * 2

        def point_block(pt, wvec, q):
            b0 = _bcast_lane(wvec, 4 * q)
            b1 = _bcast_lane(wvec, 4 * q + 1)
            b2 = _bcast_lane(wvec, 4 * q + 2)
            b3 = _bcast_lane(wvec, 4 * q + 3)
            ptv = jnp.full((16,), pt, jnp.int32)
            for cc in range(8):
                sl = pl.ds(cc * 16, 16)
                w0v = rowsv[pt, sl]
                w1v = rowsv[pt + 49, sl]
                w2v = rowsv[pt + 98, sl]
                w3v = rowsv[pt + 147, sl]

                def lo(w):
                    return plsc.bitcast(w << 16, jnp.float32)

                def hi(w):
                    return plsc.bitcast(w & himask, jnp.float32)

                ae = ((lo(w0v) * b0 + lo(w1v) * b1)
                      + (lo(w2v) * b2 + lo(w3v) * b3))
                ao = ((hi(w0v) * b0 + hi(w1v) * b1)
                      + (hi(w2v) * b2 + hi(w3v) * b3))
                cols = two_iota + (32 * cc)
                plsc.store_scatter(outv, [ptv, cols], ae)
                plsc.store_scatter(outv, [ptv, cols + 1], ao)

        def grp_body(g, carry2):
            wvec = wv[pl.ds(g * 16, 16)]
            for q in range(4):
                point_block(g * 4 + q, wvec, q)
            return carry2

        lax.fori_loop(0, 12, grp_body, 0)
        point_block(48, wv[pl.ds(192, 16)], 0)
        pltpu.async_copy(outv, out.at[pl.ds((base + r) * 56, 56)], semo)

    gen(0, idx0, w0)
    fire(idx0, rows0, semg0)

    def pair_body(i, carry):
        a = 2 * i
        gen(a + 1, idx1, w1)
        fire(idx1, rows1, semg1)
        drain(idx0, rows0, semg0)
        compute_out(a, rows0, w0, first=True)

        @pl.when(i < (_RPW // 2 - 1))
        def _():
            gen(a + 2, idx0, w0)
            fire(idx0, rows0, semg0)

        drain(idx1, rows1, semg1)
        compute_out(a + 1, rows1, w1, first=False)
        return carry

    lax.fori_loop(0, _RPW // 2, pair_body, 0)
    pltpu.make_async_copy(outv, out.at[pl.ds(0, 56)], semo).wait()


def kernel(feats_0, feats_1, feats_2, feats_3, rois):
    feats = (feats_0, feats_1, feats_2, feats_3)
    cast = [lax.optimization_barrier(f[0].astype(jnp.bfloat16))
            for f in feats]
    tabs = [c.transpose(1, 2, 0).reshape(-1, _C) for c in cast]
    tabs.append(jnp.zeros((_TABLE_ROWS - 53125, _C), jnp.bfloat16))
    table = lax.optimization_barrier(jnp.concatenate(tabs, axis=0))
    table = lax.bitcast_convert_type(
        table.reshape(_TABLE_ROWS, 128, 2), jnp.int32)
    pad = jnp.zeros((_N_PAD - _N_ROI,), jnp.float32)
    cols = [jnp.concatenate([rois[:, i], pad]) for i in (1, 2, 3, 4)]
    out_rows = _sc_roi_align(table, cols[0], cols[1], cols[2], cols[3])
    out = out_rows.reshape(_N_PAD, 56, _C)[: _N_ROI, :49]
    out = out.reshape(_N_ROI, _OUT, _OUT, _C)
    return jnp.transpose(out, (0, 3, 1, 2))


# speedup vs baseline: 2.7048x; 1.3197x over previous
import functools

import jax
import jax.numpy as jnp
import numpy as np
from jax import lax
from jax.experimental import pallas as pl
from jax.experimental.pallas import tpu as pltpu
from jax.experimental.pallas import tpu_sc as plsc

_OUT = 7
_C = 256
_HW = (200, 100, 50, 25)
_BASES = (0, 40000, 50000, 52500)
_TABLE_ROWS = 53152
_N_ROI = 1000
_N_PAD = 1024
_NC = 2
_NS = 16
_NW = _NC * _NS
_RPW = _N_PAD // _NW
_T2 = tuple(float((56.0 * (2.0 ** k - 1e-6)) ** 2) for k in (1, 2, 3))

_mesh = plsc.VectorSubcoreMesh(core_axis_name="c", subcore_axis_name="s")


def _sel_lvl(lvl, vals, dtype):
    v = jnp.full((16,), vals[3], dtype)
    v = jnp.where(lvl == 2, jnp.full((16,), vals[2], dtype), v)
    v = jnp.where(lvl == 1, jnp.full((16,), vals[1], dtype), v)
    v = jnp.where(lvl == 0, jnp.full((16,), vals[0], dtype), v)
    return v


_GDN = lax.GatherDimensionNumbers(
    offset_dims=(), collapsed_slice_dims=(0,), start_index_map=(0,))


def _bcast_lane(vec, j):
    lane = jnp.full((16, 1), j, jnp.int32)
    return lax.gather(vec, lane, _GDN, (1,),
                      mode=lax.GatherScatterMode.PROMISE_IN_BOUNDS)


@functools.partial(
    pl.kernel,
    out_type=jax.ShapeDtypeStruct((_N_PAD * 56, _C), jnp.float32),
    mesh=_mesh,
    scratch_types=[
        pltpu.VMEM((_RPW,), jnp.float32),
        pltpu.VMEM((_RPW,), jnp.float32),
        pltpu.VMEM((_RPW,), jnp.float32),
        pltpu.VMEM((_RPW,), jnp.float32),
        pltpu.VMEM((208,), jnp.int32),
        pltpu.VMEM((208,), jnp.float32),
        pltpu.VMEM((208,), jnp.int32),
        pltpu.VMEM((208,), jnp.float32),
        pltpu.VMEM((200, _C), jnp.float32),
        pltpu.VMEM((200, _C), jnp.float32),
        pltpu.VMEM((56, _C), jnp.float32),
        pltpu.SemaphoreType.DMA,
        pltpu.SemaphoreType.DMA,
        pltpu.SemaphoreType.DMA,
    ],
    compiler_params=pltpu.CompilerParams(needs_layout_passes=False),
)
def _sc_roi_align(table, rx1, ry1, rx2, ry2, out,
                  x1v, y1v, x2v, y2v, idx0, w0, idx1, w1,
                  rows0, rows1, outv, semg0, semg1, semo):
    wid = lax.axis_index("s") * _NC + lax.axis_index("c")
    base = wid * _RPW
    pltpu.sync_copy(rx1.at[pl.ds(base, _RPW)], x1v)
    pltpu.sync_copy(ry1.at[pl.ds(base, _RPW)], y1v)
    pltpu.sync_copy(rx2.at[pl.ds(base, _RPW)], x2v)
    pltpu.sync_copy(ry2.at[pl.ds(base, _RPW)], y2v)
    for st in (idx0, idx1):
        st[pl.ds(192, 16)] = jnp.zeros((16,), jnp.int32)
    for st in (w0, w1):
        st[pl.ds(192, 16)] = jnp.zeros((16,), jnp.float32)

    def gen(r, idxv, wv):
        c16 = r // 16
        j = r - c16 * 16
        bx1 = _bcast_lane(x1v[pl.ds(c16 * 16, 16)], j)
        by1 = _bcast_lane(y1v[pl.ds(c16 * 16, 16)], j)
        bx2 = _bcast_lane(x2v[pl.ds(c16 * 16, 16)], j)
        by2 = _bcast_lane(y2v[pl.ds(c16 * 16, 16)], j)
        area = (bx2 - bx1 + 1.0) * (by2 - by1 + 1.0)
        lvl = (jnp.where(area >= _T2[0], 1, 0)
               + jnp.where(area >= _T2[1], 1, 0)
               + jnp.where(area >= _T2[2], 1, 0))
        ss = _sel_lvl(lvl, (0.25, 0.125, 0.0625, 0.03125), jnp.float32)
        dmax = _sel_lvl(lvl, (199.0, 99.0, 49.0, 24.0), jnp.float32)
        wrow = _sel_lvl(lvl, _HW, jnp.int32)
        bofs = _sel_lvl(lvl, _BASES, jnp.int32)
        x1s = bx1 * ss
        y1s = by1 * ss
        roiw = jnp.maximum(bx2 * ss - x1s, 1.0)
        roih = jnp.maximum(by2 * ss - y1s, 1.0)
        binw = roiw / 7.0
        binh = roih / 7.0
        zero = jnp.zeros((16,), jnp.float32)
        for k in range(4):
            p = lax.iota(jnp.int32, 16) + (16 * k)
            pf = p.astype(jnp.float32)
            py = ((pf + 0.5) * (1.0 / 7.0)).astype(jnp.int32)
            px = p - py * 7
            yy = y1s + (py.astype(jnp.float32) + 0.5) * binh
            xx = x1s + (px.astype(jnp.float32) + 0.5) * binw
            yy = jnp.minimum(jnp.maximum(yy, zero), dmax)
            xx = jnp.minimum(jnp.maximum(xx, zero), dmax)
            y0 = yy.astype(jnp.int32)
            x0 = xx.astype(jnp.int32)
            ly = yy - y0.astype(jnp.float32)
            lx = xx - x0.astype(jnp.float32)
            hy = 1.0 - ly
            hx = 1.0 - lx
            i00 = bofs + y0 * wrow + x0
            mask = p < 49
            pc = jnp.where(mask, p, 0)
            taps = (i00, i00 + 1, i00 + wrow, i00 + wrow + 1)
            wts = (hy * hx, hy * lx, ly * hx, ly * lx)
            for t in range(4):
                plsc.store_scatter(idxv, [pc + (49 * t)], taps[t], mask=mask)
                plsc.store_scatter(wv, [pc * 4 + t], wts[t], mask=mask)

    def fire(idxv, rowsv, sem):
        pltpu.async_copy(table.at[idxv.at[pl.ds(0, 96)]],
                         rowsv.at[pl.ds(0, 96)], sem)
        pltpu.async_copy(table.at[idxv.at[pl.ds(96, 104)]],
                         rowsv.at[pl.ds(96, 104)], sem)

    def drain(idxv, rowsv, sem):
        pltpu.make_async_copy(table.at[idxv.at[pl.ds(0, 96)]],
                              rowsv.at[pl.ds(0, 96)], sem).wait()
        pltpu.make_async_copy(table.at[idxv.at[pl.ds(96, 104)]],
                              rowsv.at[pl.ds(96, 104)], sem).wait()

    def compute_out(r, rowsv, wv, first):
        if first:
            pl.when(r > 0)(lambda: pltpu.make_async_copy(
                outv, out.at[pl.ds(0, 56)], semo).wait())
        else:
            pltpu.make_async_copy(outv, out.at[pl.ds(0, 56)], semo).wait()

        def point_block(pt, wvec, q):
            b0 = _bcast_lane(wvec, 4 * q)
            b1 = _bcast_lane(wvec, 4 * q + 1)
            b2 = _bcast_lane(wvec, 4 * q + 2)
            b3 = _bcast_lane(wvec, 4 * q + 3)
            for cc in range(16):
                sl = pl.ds(cc * 16, 16)
                a01 = rowsv[pt, sl] * b0 + rowsv[pt + 49, sl] * b1
                a23 = rowsv[pt + 98, sl] * b2 + rowsv[pt + 147, sl] * b3
                outv[pt, sl] = a01 + a23

        def grp_body(g, carry2):
            wvec = wv[pl.ds(g * 16, 16)]
            for q in range(4):
                point_block(g * 4 + q, wvec, q)
            return carry2

        lax.fori_loop(0, 12, grp_body, 0)
        point_block(48, wv[pl.ds(192, 16)], 0)
        pltpu.async_copy(outv, out.at[pl.ds((base + r) * 56, 56)], semo)

    gen(0, idx0, w0)
    fire(idx0, rows0, semg0)

    def pair_body(i, carry):
        a = 2 * i
        gen(a + 1, idx1, w1)
        fire(idx1, rows1, semg1)
        drain(idx0, rows0, semg0)
        compute_out(a, rows0, w0, first=True)

        @pl.when(i < (_RPW // 2 - 1))
        def _():
            gen(a + 2, idx0, w0)
            fire(idx0, rows0, semg0)

        drain(idx1, rows1, semg1)
        compute_out(a + 1, rows1, w1, first=False)
        return carry

    lax.fori_loop(0, _RPW // 2, pair_body, 0)
    pltpu.make_async_copy(outv, out.at[pl.ds(0, 56)], semo).wait()


def kernel(feats_0, feats_1, feats_2, feats_3, rois):
    feats = (feats_0, feats_1, feats_2, feats_3)
    tabs = [f[0].transpose(1, 2, 0).reshape(-1, _C) for f in feats]
    tabs.append(jnp.zeros((_TABLE_ROWS - 53125, _C), jnp.float32))
    table = jnp.concatenate(tabs, axis=0)
    pad = jnp.zeros((_N_PAD - _N_ROI,), jnp.float32)
    cols = [jnp.concatenate([rois[:, i], pad]) for i in (1, 2, 3, 4)]
    out_rows = _sc_roi_align(table, cols[0], cols[1], cols[2], cols[3])
    out = out_rows.reshape(_N_PAD, 56, _C)[: _N_ROI, :49]
    out = out.reshape(_N_ROI, _OUT, _OUT, _C)
    return jnp.transpose(out, (0, 3, 1, 2))
